# baseline (speedup 1.0000x reference)
import functools
import math

import jax
import jax.numpy as jnp
from jax import lax
from jax.experimental import pallas as pl
from jax.experimental.pallas import tpu as pltpu
from jax.experimental.pallas import tpu_sc as plsc

VOCAB = 1000000
D = 64
BATCH = 4096
SEQ = 200
SCALE = math.sqrt(D)

NC = 2
NS = 16
LANES = 16
NW = NC * NS

TOTAL = BATCH * SEQ
PER_W = TOTAL // NW
CHUNK = 128
KPG = 2
G = CHUNK * KPG
NG = PER_W // G
NCHUNK = PER_W // CHUNK


def _emb_body(x_hbm, tab_hbm, out_hbm, idx_v, rows_v, gsem0, gsem1):
    wid = lax.axis_index("s") * NC + lax.axis_index("c")
    gsems = (gsem0, gsem1)
    pltpu.sync_copy(x_hbm.at[wid], idx_v)

    def gather_desc(g, b, j):
        return pltpu.make_async_copy(
            tab_hbm.at[idx_v.at[g * KPG + j]],
            rows_v.at[b, pl.ds(j * CHUNK, CHUNK)],
            gsems[b],
        )

    def fire(g, b):
        for j in range(KPG):
            gather_desc(g, b, j).start()

    def process(g, b, do_fire):
        for j in range(KPG):
            gather_desc(g, b, j).wait()

        def mrow(r, _):
            for q in range(D // LANES):
                sl = pl.ds(q * LANES, LANES)
                rows_v[b, r, sl] = rows_v[b, r, sl] * SCALE
            return ()

        lax.fori_loop(0, G, mrow, (), unroll=8)
        pltpu.sync_copy(
            rows_v.at[b],
            out_hbm.at[pl.ds(wid * PER_W + g * G, G)])
        if do_fire:
            fire(g + 2, b)

    fire(0, 0)
    fire(1, 1)

    def step(t, _):
        process(2 * t, 0, True)
        process(2 * t + 1, 1, True)
        return ()

    lax.fori_loop(0, NG // 2 - 1, step, ())
    process(NG - 2, 0, False)
    process(NG - 1, 1, False)


@jax.jit
def _emb_lookup(x2d, tab128):
    mesh = plsc.VectorSubcoreMesh(core_axis_name="c", subcore_axis_name="s")
    k = functools.partial(
        pl.kernel,
        out_type=jax.ShapeDtypeStruct((TOTAL, 128), jnp.float32),
        mesh=mesh,
        scratch_types=[
            pltpu.VMEM((NCHUNK, CHUNK), jnp.int32),
            pltpu.VMEM((2, G, 128), jnp.float32),
            pltpu.SemaphoreType.DMA,
            pltpu.SemaphoreType.DMA,
        ],
        compiler_params=pltpu.CompilerParams(use_tc_tiling_on_sc=False),
    )(_emb_body)
    return k(x2d, tab128)


def kernel(x, emb_table):
    x2d = x.astype(jnp.int32).reshape(NW, NCHUNK, CHUNK)
    tab128 = jnp.pad(emb_table, ((0, 0), (0, 128 - D)))
    out = _emb_lookup(x2d, tab128)
    return out[:, :D].reshape(BATCH, SEQ, D)


# speedup vs baseline: 1.0904x; 1.0904x over previous
import functools
import math

import jax
import jax.numpy as jnp
from jax import lax
from jax.experimental import pallas as pl
from jax.experimental.pallas import tpu as pltpu
from jax.experimental.pallas import tpu_sc as plsc

VOCAB = 1000000
D = 64
BATCH = 4096
SEQ = 200
SCALE = math.sqrt(D)

NC = 2
NS = 16
LANES = 16
NW = NC * NS

TOTAL = BATCH * SEQ
PER_W = TOTAL // NW
CHUNK = 128
KPG = 4
G = CHUNK * KPG
NG = PER_W // G
NCHUNK = PER_W // CHUNK


def _emb_body(x_hbm, tab_hbm, out_hbm, idx_v, rows_v, gsem0, gsem1):
    wid = lax.axis_index("s") * NC + lax.axis_index("c")
    gsems = (gsem0, gsem1)
    pltpu.sync_copy(x_hbm.at[wid], idx_v)

    def gather_desc(g, b, j):
        return pltpu.make_async_copy(
            tab_hbm.at[idx_v.at[g * KPG + j]],
            rows_v.at[b, pl.ds(j * CHUNK, CHUNK)],
            gsems[b],
        )

    def fire(g, b):
        for j in range(KPG):
            gather_desc(g, b, j).start()

    def process(g, b, do_fire):
        for j in range(KPG):
            gather_desc(g, b, j).wait()

        def mrow(r, _):
            for q in range(D // LANES):
                sl = pl.ds(q * LANES, LANES)
                rows_v[b, r, sl] = rows_v[b, r, sl] * SCALE
            return ()

        lax.fori_loop(0, G, mrow, (), unroll=8)
        pltpu.sync_copy(
            rows_v.at[b],
            out_hbm.at[pl.ds(wid * PER_W + g * G, G), pl.ds(0, D)])
        if do_fire:
            fire(g + 2, b)

    fire(0, 0)
    fire(1, 1)

    def step(t, _):
        process(2 * t, 0, True)
        process(2 * t + 1, 1, True)
        return ()

    lax.fori_loop(0, NG // 2 - 1, step, ())
    process(NG - 2, 0, False)
    process(NG - 1, 1, False)


@jax.jit
def _emb_lookup(x2d, emb_table):
    mesh = plsc.VectorSubcoreMesh(core_axis_name="c", subcore_axis_name="s")
    k = functools.partial(
        pl.kernel,
        out_type=jax.ShapeDtypeStruct((TOTAL, 128), jnp.float32),
        mesh=mesh,
        scratch_types=[
            pltpu.VMEM((NCHUNK, CHUNK), jnp.int32),
            pltpu.VMEM((2, G, D), jnp.float32),
            pltpu.SemaphoreType.DMA,
            pltpu.SemaphoreType.DMA,
        ],
        compiler_params=pltpu.CompilerParams(use_tc_tiling_on_sc=False),
    )(_emb_body)
    return k(x2d, emb_table)


def kernel(x, emb_table):
    x2d = x.astype(jnp.int32).reshape(NW, NCHUNK, CHUNK)
    out = _emb_lookup(x2d, emb_table)
    return out[:, :D].reshape(BATCH, SEQ, D)
